# baseline (speedup 1.0000x reference)
import functools

import jax
import jax.numpy as jnp
from jax import lax
from jax.experimental import pallas as pl
from jax.experimental.pallas import tpu as pltpu
from jax.experimental.pallas import tpu_sc as plsc

D_MODEL = 1024
N_IDS = 4 * 4096

_NC, _NS = 2, 16
_NW = _NC * _NS
_PER_W = N_IDS // _NW

_SPLIT_B = 80
_SPLIT_A = _PER_W - _SPLIT_B
_CHUNK = 16
_NCHUNK = _SPLIT_A // _CHUNK
_BGROUPS = _SPLIT_B // 16


def _embed_body(
    table_hbm, idx_hbm, out_hbm, idx_v, rows0, rows1, rows_sp,
    gsem0, gsem1, ssem0, ssem1, bsem, osem
):
    wid = lax.axis_index("s") * _NC + lax.axis_index("c")
    sid = lax.axis_index("s")
    base = wid * _PER_W
    pltpu.sync_copy(idx_hbm.at[pl.ds(base, _PER_W)], idx_v)

    def bstep(g, carry):
        vec = idx_v[pl.ds(_SPLIT_A + g * 16, 16)]
        for j in range(16):
            pltpu.async_copy(
                table_hbm.at[vec[j]], rows_sp.at[sid, g * 16 + j], bsem
            )
        return carry

    lax.fori_loop(0, _BGROUPS, bstep, 0)

    bufs = (rows0, rows1)
    gsems = (gsem0, gsem1)
    ssems = (ssem0, ssem1)
    gcp = [
        pltpu.async_copy(table_hbm.at[idx_v.at[pl.ds(0, _CHUNK)]], rows0, gsem0),
        None,
    ]
    scp = [None, None]
    for c in range(_NCHUNK):
        cur = c % 2
        nxt = 1 - cur
        if c + 1 < _NCHUNK:
            if scp[nxt] is not None:
                scp[nxt].wait()
            gcp[nxt] = pltpu.async_copy(
                table_hbm.at[idx_v.at[pl.ds((c + 1) * _CHUNK, _CHUNK)]],
                bufs[nxt],
                gsems[nxt],
            )
        gcp[cur].wait()
        scp[cur] = pltpu.async_copy(
            bufs[cur], out_hbm.at[pl.ds(base + c * _CHUNK, _CHUNK)], ssems[cur]
        )

    pltpu.make_async_copy(
        table_hbm.at[pl.ds(0, _SPLIT_B)], rows_sp.at[sid], bsem
    ).wait()
    pltpu.async_copy(
        rows_sp.at[sid], out_hbm.at[pl.ds(base + _SPLIT_A, _SPLIT_B)], osem
    ).wait()

    for s in scp:
        if s is not None:
            s.wait()


@jax.jit
def _embed_lookup(table, ids):
    mesh = plsc.VectorSubcoreMesh(core_axis_name="c", subcore_axis_name="s")
    run = pl.kernel(
        _embed_body,
        mesh=mesh,
        out_type=jax.ShapeDtypeStruct((N_IDS, D_MODEL), jnp.float32),
        scratch_types=[
            pltpu.VMEM((_PER_W,), jnp.int32),
            pltpu.VMEM((_CHUNK, D_MODEL), jnp.float32),
            pltpu.VMEM((_CHUNK, D_MODEL), jnp.float32),
            pltpu.VMEM_SHARED((_NS, _SPLIT_B, D_MODEL), jnp.float32),
            pltpu.SemaphoreType.DMA,
            pltpu.SemaphoreType.DMA,
            pltpu.SemaphoreType.DMA,
            pltpu.SemaphoreType.DMA,
            pltpu.SemaphoreType.DMA,
            pltpu.SemaphoreType.DMA,
        ],
    )
    return run(table, ids)


def kernel(input_ids, is_node, node_features, edge_index, mapping, embed_weight):
    ids = input_ids.reshape(-1)
    out = _embed_lookup(embed_weight, ids)
    return out.reshape(input_ids.shape[0], input_ids.shape[1], D_MODEL)
